# baseline (speedup 1.0000x reference)
import functools

import jax
import jax.numpy as jnp
import numpy as np
from jax import lax
from jax.experimental import pallas as pl
from jax.experimental.pallas import tpu as pltpu, tpu_sc as plsc

_NS, _L = 16, 16
_I32_MIN = np.int32(-2147483648)


def _build(B, S, D):
    chunks = S // _L
    mesh = plsc.VectorSubcoreMesh(core_axis_name="c", subcore_axis_name="s",
                                  num_cores=1, num_subcores=_NS)

    @functools.partial(
        pl.kernel,
        out_type=jax.ShapeDtypeStruct((B,), jnp.float32),
        mesh=mesh,
        scratch_types=[
            pltpu.VMEM((B, S), jnp.int32),
            pltpu.VMEM((B, 128), jnp.float32),
            pltpu.VMEM((_L,), jnp.float32),
            [pltpu.SemaphoreType.DMA] * 4,
            pltpu.SemaphoreType.DMA,
        ],
        compiler_params=pltpu.CompilerParams(needs_layout_passes=False),
    )
    def sc_kernel(lhs_hbm, ids_hbm, out_hbm, ids_v, win_v, outv_v, row_sems,
                  win_sem):
        cid = lax.axis_index("c")
        sid = lax.axis_index("s")
        iota = lax.broadcasted_iota(jnp.int32, (_L,), 0)

        @pl.when(jnp.logical_and(cid == 0, sid == 0))
        def _():
            row_copies = [
                pltpu.async_copy(ids_hbm.at[b], ids_v.at[b], row_sems[b])
                for b in range(B)
            ]
            idxs = []
            for b in range(B):
                row_copies[b].wait()

                def body(c, carry):
                    acc, inv = carry
                    v = ids_v[b, pl.ds(c * _L, _L)]
                    return jnp.maximum(acc, (v << 11) | inv), inv - _L

                acc, _ = lax.fori_loop(
                    0, chunks, body,
                    (jnp.full((_L,), _I32_MIN, jnp.int32), (S - 1) - iota))
                k = jnp.max(acc)
                idxs.append((S - 1) - (k & (S - 1)))
            win_copies = []
            for b in range(B):
                cs = (idxs[b] // 128) * 128
                win_copies.append(pltpu.async_copy(
                    lhs_hbm.at[b, idxs[b], pl.ds(cs, 128)], win_v.at[b],
                    win_sem))
            for cp in win_copies:
                cp.wait()
            outv = jnp.zeros((_L,), jnp.float32)
            for b in range(B):
                c16 = ((idxs[b] % 128) // _L) * _L
                lane = idxs[b] % _L
                w16 = win_v[b, pl.ds(c16, _L)]
                val = jnp.sum(jnp.where(iota == lane, w16, 0.0))
                outv = jnp.where(iota == b, val * val, outv)
            outv_v[...] = outv
            pltpu.sync_copy(outv_v.at[pl.ds(0, B)], out_hbm)

    return sc_kernel


def kernel(last_hidden_state, input_ids):
    B, S, D = last_hidden_state.shape
    return _build(B, S, D)(last_hidden_state, input_ids.astype(jnp.int32))


# speedup vs baseline: 1.0129x; 1.0129x over previous
import functools

import jax
import jax.numpy as jnp
import numpy as np
from jax import lax
from jax.experimental import pallas as pl
from jax.experimental.pallas import tpu as pltpu, tpu_sc as plsc

_NS, _L = 16, 16
_I32_MIN = np.int32(-2147483648)


def _build(B, S, D):
    chunks = S // _L
    mesh = plsc.VectorSubcoreMesh(core_axis_name="c", subcore_axis_name="s",
                                  num_cores=1, num_subcores=1)

    @functools.partial(
        pl.kernel,
        out_type=jax.ShapeDtypeStruct((B,), jnp.float32),
        mesh=mesh,
        scratch_types=[
            pltpu.VMEM((B, S), jnp.int32),
            pltpu.VMEM((B, 128), jnp.float32),
            pltpu.VMEM((_L,), jnp.float32),
            [pltpu.SemaphoreType.DMA] * 4,
            pltpu.SemaphoreType.DMA,
        ],
        compiler_params=pltpu.CompilerParams(needs_layout_passes=False),
    )
    def sc_kernel(lhs_hbm, ids_hbm, out_hbm, ids_v, win_v, outv_v, row_sems,
                  win_sem):
        cid = lax.axis_index("c")
        sid = lax.axis_index("s")
        iota = lax.broadcasted_iota(jnp.int32, (_L,), 0)

        @pl.when(jnp.logical_and(cid == 0, sid == 0))
        def _():
            row_copies = [
                pltpu.async_copy(ids_hbm.at[b], ids_v.at[b], row_sems[b])
                for b in range(B)
            ]
            idxs = []
            for b in range(B):
                row_copies[b].wait()

                def body(c, carry):
                    acc, inv = carry
                    v = ids_v[b, pl.ds(c * _L, _L)]
                    return jnp.maximum(acc, (v << 11) | inv), inv - _L

                acc, _ = lax.fori_loop(
                    0, chunks, body,
                    (jnp.full((_L,), _I32_MIN, jnp.int32), (S - 1) - iota))
                k = jnp.max(acc)
                idxs.append((S - 1) - (k & (S - 1)))
            win_copies = []
            for b in range(B):
                cs = (idxs[b] // 128) * 128
                win_copies.append(pltpu.async_copy(
                    lhs_hbm.at[b, idxs[b], pl.ds(cs, 128)], win_v.at[b],
                    win_sem))
            for cp in win_copies:
                cp.wait()
            outv = jnp.zeros((_L,), jnp.float32)
            for b in range(B):
                c16 = ((idxs[b] % 128) // _L) * _L
                lane = idxs[b] % _L
                w16 = win_v[b, pl.ds(c16, _L)]
                val = jnp.sum(jnp.where(iota == lane, w16, 0.0))
                outv = jnp.where(iota == b, val * val, outv)
            outv_v[...] = outv
            pltpu.sync_copy(outv_v.at[pl.ds(0, B)], out_hbm)

    return sc_kernel


def kernel(last_hidden_state, input_ids):
    B, S, D = last_hidden_state.shape
    return _build(B, S, D)(last_hidden_state, input_ids.astype(jnp.int32))


# speedup vs baseline: 1.0159x; 1.0029x over previous
import functools

import jax
import jax.numpy as jnp
import numpy as np
from jax import lax
from jax.experimental import pallas as pl
from jax.experimental.pallas import tpu as pltpu, tpu_sc as plsc

_L = 16
_I32_MIN = np.int32(-2147483648)


def _build(B, S, D):
    chunks = S // _L
    mesh = plsc.VectorSubcoreMesh(core_axis_name="c", subcore_axis_name="s",
                                  num_cores=1, num_subcores=B)

    @functools.partial(
        pl.kernel,
        out_type=jax.ShapeDtypeStruct((B, _L), jnp.float32),
        mesh=mesh,
        scratch_types=[
            pltpu.VMEM((S,), jnp.int32),
            pltpu.VMEM((128,), jnp.float32),
            pltpu.VMEM((_L,), jnp.float32),
        ],
        compiler_params=pltpu.CompilerParams(
            needs_layout_passes=False,
            disable_bounds_checks=True,
            disable_semaphore_checks=True,
            skip_device_barrier=True,
        ),
    )
    def sc_kernel(lhs_hbm, ids_hbm, out_hbm, ids_v, win_v, outv_v):
        cid = lax.axis_index("c")
        sid = lax.axis_index("s")
        iota = lax.broadcasted_iota(jnp.int32, (_L,), 0)

        @pl.when(jnp.logical_and(cid == 0, sid < B))
        def _():
            b = sid
            pltpu.sync_copy(ids_hbm.at[b], ids_v)
            inv = (S - 1) - iota
            acc = jnp.full((_L,), _I32_MIN, jnp.int32)
            for c in range(chunks):
                v = ids_v[pl.ds(c * _L, _L)]
                acc = jnp.maximum(acc, (v << 11) | inv)
                inv = inv - _L
            k = jnp.max(acc)
            idx = (S - 1) - (k & (S - 1))
            cs = (idx // 128) * 128
            pltpu.sync_copy(lhs_hbm.at[b, idx, pl.ds(cs, 128)], win_v)
            c16 = ((idx % 128) // _L) * _L
            lane = idx % _L
            w16 = win_v[pl.ds(c16, _L)]
            val = jnp.sum(jnp.where(iota == lane, w16, 0.0))
            outv_v[...] = jnp.zeros((_L,), jnp.float32) + val * val
            pltpu.sync_copy(outv_v, out_hbm.at[b])

    return sc_kernel


def kernel(last_hidden_state, input_ids):
    B, S, D = last_hidden_state.shape
    out = _build(B, S, D)(last_hidden_state, input_ids.astype(jnp.int32))
    return out[:, 0]


# speedup vs baseline: 1.0376x; 1.0214x over previous
import functools

import jax
import jax.numpy as jnp
import numpy as np
from jax import lax
from jax.experimental import pallas as pl
from jax.experimental.pallas import tpu as pltpu, tpu_sc as plsc

_L = 16
_I32_MIN = np.int32(-2147483648)


def _build(B, S, D):
    chunks = S // _L
    mesh = plsc.VectorSubcoreMesh(core_axis_name="c", subcore_axis_name="s",
                                  num_cores=1, num_subcores=B)

    @functools.partial(
        pl.kernel,
        out_type=jax.ShapeDtypeStruct((B, _L), jnp.float32),
        mesh=mesh,
        scratch_types=[
            pltpu.VMEM((S,), jnp.int32),
            pltpu.VMEM((128,), jnp.float32),
            pltpu.VMEM((_L,), jnp.float32),
        ],
        compiler_params=pltpu.CompilerParams(
            needs_layout_passes=False,
            disable_bounds_checks=True,
            disable_semaphore_checks=True,
            skip_device_barrier=True,
        ),
    )
    def sc_kernel(lhs_hbm, ids_hbm, out_hbm, ids_v, win_v, outv_v):
        cid = lax.axis_index("c")
        sid = lax.axis_index("s")
        iota = lax.broadcasted_iota(jnp.int32, (_L,), 0)

        @pl.when(jnp.logical_and(cid == 0, sid < B))
        def _():
            b = sid
            pltpu.sync_copy(ids_hbm.at[b], ids_v)
            def body(c, carry):
                acc, inv = carry
                v = ids_v[pl.ds(c * _L, _L)]
                return jnp.maximum(acc, (v << 11) | inv), inv - _L

            acc, _ = lax.fori_loop(
                0, chunks, body,
                (jnp.full((_L,), _I32_MIN, jnp.int32), (S - 1) - iota))
            k = jnp.max(acc)
            idx = (S - 1) - (k & (S - 1))
            cs = (idx // 128) * 128
            pltpu.sync_copy(lhs_hbm.at[b, idx, pl.ds(cs, 128)], win_v)
            c16 = ((idx % 128) // _L) * _L
            lane = idx % _L
            w16 = win_v[pl.ds(c16, _L)]
            val = jnp.sum(jnp.where(iota == lane, w16, 0.0))
            outv_v[...] = jnp.zeros((_L,), jnp.float32) + val * val
            pltpu.sync_copy(outv_v, out_hbm.at[b])

    return sc_kernel


def kernel(last_hidden_state, input_ids):
    B, S, D = last_hidden_state.shape
    out = _build(B, S, D)(last_hidden_state, input_ids.astype(jnp.int32))
    return out[:, 0]


# speedup vs baseline: 1.0631x; 1.0245x over previous
import functools

import jax
import jax.numpy as jnp
import numpy as np
from jax import lax
from jax.experimental import pallas as pl
from jax.experimental.pallas import tpu as pltpu, tpu_sc as plsc

_L = 16
_I32_MIN = np.int32(-2147483648)


def _build(B, S, D):
    chunks = S // _L
    mesh = plsc.VectorSubcoreMesh(core_axis_name="c", subcore_axis_name="s",
                                  num_cores=1, num_subcores=B)

    @functools.partial(
        pl.kernel,
        out_type=jax.ShapeDtypeStruct((B, _L), jnp.float32),
        mesh=mesh,
        scratch_types=[
            pltpu.VMEM((S,), jnp.int32),
            pltpu.VMEM((128,), jnp.float32),
            pltpu.VMEM((_L,), jnp.float32),
        ],
        compiler_params=pltpu.CompilerParams(
            needs_layout_passes=False,
            disable_bounds_checks=True,
            disable_semaphore_checks=True,
            skip_device_barrier=True,
        ),
    )
    def sc_kernel(lhs_hbm, ids_hbm, out_hbm, ids_v, win_v, outv_v):
        cid = lax.axis_index("c")
        sid = lax.axis_index("s")
        iota = lax.broadcasted_iota(jnp.int32, (_L,), 0)

        @pl.when(jnp.logical_and(cid == 0, sid < B))
        def _():
            b = sid
            pltpu.sync_copy(ids_hbm.at[b], ids_v)
            unroll = 8

            def body(c, carry):
                acc, inv = carry
                for u in range(unroll):
                    v = ids_v[pl.ds((c * unroll + u) * _L, _L)]
                    acc = jnp.maximum(acc, (v << 11) | inv)
                    inv = inv - _L
                return acc, inv

            acc, _ = lax.fori_loop(
                0, chunks // unroll, body,
                (jnp.full((_L,), _I32_MIN, jnp.int32), (S - 1) - iota))
            k = jnp.max(acc)
            idx = (S - 1) - (k & (S - 1))
            cs = (idx // 128) * 128
            pltpu.sync_copy(lhs_hbm.at[b, idx, pl.ds(cs, 128)], win_v)
            c16 = ((idx % 128) // _L) * _L
            lane = idx % _L
            w16 = win_v[pl.ds(c16, _L)]
            val = jnp.sum(jnp.where(iota == lane, w16, 0.0))
            outv_v[...] = jnp.zeros((_L,), jnp.float32) + val * val
            pltpu.sync_copy(outv_v, out_hbm.at[b])

    return sc_kernel


def kernel(last_hidden_state, input_ids):
    B, S, D = last_hidden_state.shape
    out = _build(B, S, D)(last_hidden_state, input_ids.astype(jnp.int32))
    return out[:, 0]
